# baseline (speedup 1.0000x reference)
import jax
import jax.numpy as jnp
from jax.experimental import pallas as pl
from jax.experimental.pallas import tpu as pltpu

NODE = 325
BATCH = 32
HID = 64
T = 12
NMAT = 5
BB = 16
F = 2 * HID


def _layer_kernel(x_ref, sup_ref, h0_ref, wg_ref, bg_ref, wc_ref, bc_ref,
                  out_ref, h_scr):
    t = pl.program_id(1)

    @pl.when(t == 0)
    def _():
        h_scr[...] = h0_ref[...]

    x3 = x_ref[0]
    h3 = h_scr[...]

    def dconv(s3, w_ref, b_ref):
        out_dim = w_ref.shape[2]
        xs = jnp.concatenate([x3, s3], axis=2).reshape(NODE, BB * F)

        def proj(m, matb):
            r = matb.reshape(NODE * BB, F)
            return jnp.dot(r, w_ref[m], preferred_element_type=jnp.float32)

        xsb = xs.astype(jnp.bfloat16)
        acc = proj(0, xsb) + b_ref[...]
        m = 1
        for s in range(2):
            supb = sup_ref[s]
            x1 = jnp.dot(supb, xsb, preferred_element_type=jnp.float32)
            x1b = x1.astype(jnp.bfloat16)
            acc = acc + proj(m, x1b)
            m += 1
            x2 = 2.0 * jnp.dot(supb, x1b, preferred_element_type=jnp.float32) - xs
            acc = acc + proj(m, x2.astype(jnp.bfloat16))
            m += 1
        return acc.reshape(NODE, BB, out_dim)

    g = jax.nn.sigmoid(dconv(h3, wg_ref, bg_ref))
    r = g[:, :, :HID]
    u = g[:, :, HID:]
    c = jnp.tanh(dconv(r * h3, wc_ref, bc_ref))
    h_new = u * h3 + (1.0 - u) * c
    h_scr[...] = h_new
    out_ref[0] = h_new


def _run_layer(xseq, supports, h0, wg, bg, wc, bc):
    nb = BATCH // BB
    return pl.pallas_call(
        _layer_kernel,
        grid=(nb, T),
        in_specs=[
            pl.BlockSpec((1, NODE, BB, HID), lambda b, t: (t, 0, b, 0)),
            pl.BlockSpec((2, NODE, NODE), lambda b, t: (0, 0, 0)),
            pl.BlockSpec((NODE, BB, HID), lambda b, t: (0, b, 0)),
            pl.BlockSpec((NMAT, F, 2 * HID), lambda b, t: (0, 0, 0)),
            pl.BlockSpec((1, 2 * HID), lambda b, t: (0, 0)),
            pl.BlockSpec((NMAT, F, HID), lambda b, t: (0, 0, 0)),
            pl.BlockSpec((1, HID), lambda b, t: (0, 0)),
        ],
        out_specs=pl.BlockSpec((1, NODE, BB, HID), lambda b, t: (t, 0, b, 0)),
        out_shape=jax.ShapeDtypeStruct((T, NODE, BATCH, HID), jnp.float32),
        scratch_shapes=[pltpu.VMEM((NODE, BB, HID), jnp.float32)],
        compiler_params=pltpu.CompilerParams(
            dimension_semantics=("arbitrary", "arbitrary")),
    )(xseq, supports, h0, wg, bg, wc, bc)


def _reorder_w(w, I):
    out_dim = w.shape[1]
    w = w.reshape(I + HID, NMAT, out_dim).transpose(1, 0, 2)
    if I < HID:
        w = jnp.concatenate(
            [w[:, :I], jnp.zeros((NMAT, HID - I, out_dim), w.dtype), w[:, I:]],
            axis=1)
    return w.astype(jnp.bfloat16)


def kernel(inputs, supports, initial_hidden_state,
           Wg0, bg0, Wc0, bc0, Wg1, bg1, Wc1, bc1):
    x0 = inputs.reshape(T, BATCH, NODE, 2).transpose(0, 2, 1, 3)
    x0 = jnp.pad(x0, ((0, 0), (0, 0), (0, 0), (0, HID - 2)))
    h0 = initial_hidden_state.reshape(2, BATCH, NODE, HID).transpose(0, 2, 1, 3)
    supports = supports.astype(jnp.bfloat16)

    out0 = _run_layer(x0, supports, h0[0],
                      _reorder_w(Wg0, 2), bg0.reshape(1, -1),
                      _reorder_w(Wc0, 2), bc0.reshape(1, -1))
    out1 = _run_layer(out0, supports, h0[1],
                      _reorder_w(Wg1, HID), bg1.reshape(1, -1),
                      _reorder_w(Wc1, HID), bc1.reshape(1, -1))

    cur = out1.transpose(0, 2, 1, 3).reshape(T, BATCH, NODE * HID)
    hfin = jnp.stack([out0[T - 1], out1[T - 1]], axis=0)
    hfin = hfin.transpose(0, 2, 1, 3).reshape(2, BATCH, NODE * HID)
    return (hfin, cur)


# speedup vs baseline: 1.2192x; 1.2192x over previous
import functools

import jax
import jax.numpy as jnp
from jax.experimental import pallas as pl
from jax.experimental.pallas import tpu as pltpu

NODE = 325
BATCH = 32
HID = 64
T = 12
NMAT = 5
BB = 16
F = 2 * HID
NP8 = 328


def _layer_kernel(out_bm, x_ref, scat_ref, sup_ref, h0_ref, wg_ref, bg_ref,
                  wc_ref, bc_ref, out_ref, h_scr):
    t = pl.program_id(1)

    @pl.when(t == 0)
    def _():
        h_scr[...] = h0_ref[...]

    x3 = x_ref[0]
    h3 = h_scr[...]

    def dconv(s3, w_ref, b_ref):
        out_dim = w_ref.shape[2]
        xs = jnp.concatenate([x3, s3], axis=2).reshape(NODE, BB * F)

        def proj(m, mat):
            r = mat.reshape(NODE * BB, F)
            return jnp.dot(r, w_ref[m], preferred_element_type=jnp.float32)

        acc = proj(0, xs) + b_ref[...]
        y = jnp.dot(scat_ref[...], xs, preferred_element_type=jnp.float32)
        for s in range(2):
            x1 = y[s * NP8:s * NP8 + NODE]
            acc = acc + proj(1 + 2 * s, x1)
            x2 = 2.0 * jnp.dot(sup_ref[s], x1,
                               preferred_element_type=jnp.float32) - xs
            acc = acc + proj(2 + 2 * s, x2)
        return acc.reshape(NODE, BB, out_dim)

    g = jax.nn.sigmoid(dconv(h3, wg_ref, bg_ref))
    r = g[:, :, :HID]
    u = g[:, :, HID:]
    c = jnp.tanh(dconv(r * h3, wc_ref, bc_ref))
    h_new = u * h3 + (1.0 - u) * c
    h_scr[...] = h_new
    if out_bm:
        out_ref[0] = jnp.swapaxes(h_new, 0, 1)
    else:
        out_ref[0] = h_new


def _run_layer(xseq, scat, supports, h0, wg, bg, wc, bc, out_bm):
    nb = BATCH // BB
    if out_bm:
        out_spec = pl.BlockSpec((1, BB, NODE, HID), lambda b, t: (t, b, 0, 0))
        out_shape = jax.ShapeDtypeStruct((T, BATCH, NODE, HID), jnp.float32)
    else:
        out_spec = pl.BlockSpec((1, NODE, BB, HID), lambda b, t: (t, 0, b, 0))
        out_shape = jax.ShapeDtypeStruct((T, NODE, BATCH, HID), jnp.float32)
    kern = functools.partial(_layer_kernel, out_bm)
    return pl.pallas_call(
        kern,
        grid=(nb, T),
        in_specs=[
            pl.BlockSpec((1, NODE, BB, HID), lambda b, t: (t, 0, b, 0)),
            pl.BlockSpec((2 * NP8, NODE), lambda b, t: (0, 0)),
            pl.BlockSpec((2, NODE, NODE), lambda b, t: (0, 0, 0)),
            pl.BlockSpec((NODE, BB, HID), lambda b, t: (0, b, 0)),
            pl.BlockSpec((NMAT, F, 2 * HID), lambda b, t: (0, 0, 0)),
            pl.BlockSpec((1, 2 * HID), lambda b, t: (0, 0)),
            pl.BlockSpec((NMAT, F, HID), lambda b, t: (0, 0, 0)),
            pl.BlockSpec((1, HID), lambda b, t: (0, 0)),
        ],
        out_specs=out_spec,
        out_shape=out_shape,
        scratch_shapes=[pltpu.VMEM((NODE, BB, HID), jnp.float32)],
        compiler_params=pltpu.CompilerParams(
            dimension_semantics=("arbitrary", "arbitrary")),
    )(xseq, scat, supports, h0, wg, bg, wc, bc)


def _reorder_w(w, I):
    out_dim = w.shape[1]
    w = w.reshape(I + HID, NMAT, out_dim).transpose(1, 0, 2)
    if I < HID:
        w = jnp.concatenate(
            [w[:, :I], jnp.zeros((NMAT, HID - I, out_dim), w.dtype), w[:, I:]],
            axis=1)
    return w


def kernel(inputs, supports, initial_hidden_state,
           Wg0, bg0, Wc0, bc0, Wg1, bg1, Wc1, bc1):
    x0 = inputs.reshape(T, BATCH, NODE, 2).transpose(0, 2, 1, 3)
    x0 = jnp.pad(x0, ((0, 0), (0, 0), (0, 0), (0, HID - 2)))
    h0 = initial_hidden_state.reshape(2, BATCH, NODE, HID).transpose(0, 2, 1, 3)
    scat = jnp.zeros((2 * NP8, NODE), jnp.float32)
    scat = scat.at[0:NODE].set(supports[0]).at[NP8:NP8 + NODE].set(supports[1])

    out0 = _run_layer(x0, scat, supports, h0[0],
                      _reorder_w(Wg0, 2), bg0.reshape(1, -1),
                      _reorder_w(Wc0, 2), bc0.reshape(1, -1), False)
    out1 = _run_layer(out0, scat, supports, h0[1],
                      _reorder_w(Wg1, HID), bg1.reshape(1, -1),
                      _reorder_w(Wc1, HID), bc1.reshape(1, -1), True)

    cur = out1.reshape(T, BATCH, NODE * HID)
    h1fin = out0[T - 1].transpose(1, 0, 2).reshape(BATCH, NODE * HID)
    hfin = jnp.stack([h1fin, out1[T - 1].reshape(BATCH, NODE * HID)], axis=0)
    return (hfin, cur)
